# baseline (speedup 1.0000x reference)
import functools

import jax
import jax.numpy as jnp
from jax import lax
from jax.experimental import pallas as pl
from jax.experimental.pallas import tpu as pltpu
from jax.experimental.pallas import tpu_sc as plsc

_T = 33408


@functools.lru_cache(maxsize=None)
def _build(B, V, D):
    info = plsc.get_sparse_core_info()
    nw = info.num_cores * info.num_subcores
    rows_per_w = D // nw
    main = (V // 128) * 128
    tail = V - main
    sizes = (_T, _T, main - 2 * _T)
    bases = (0, _T, 2 * _T)
    mesh = plsc.VectorSubcoreMesh(core_axis_name="c", subcore_axis_name="s")

    @functools.partial(
        pl.kernel,
        mesh=mesh,
        out_type=jax.ShapeDtypeStruct((D, B), jnp.float32),
        compiler_params=pltpu.CompilerParams(needs_layout_passes=False),
        scratch_types=[
            pltpu.VMEM((B,), jnp.int32),
            pltpu.VMEM((_T,), jnp.float32),
            pltpu.VMEM((_T,), jnp.float32),
            pltpu.VMEM((tail,), jnp.float32),
            pltpu.VMEM((tail,), jnp.float32),
            pltpu.VMEM((B,), jnp.float32),
            pltpu.VMEM((B,), jnp.float32),
            pltpu.SemaphoreType.DMA,
            pltpu.SemaphoreType.DMA,
            pltpu.SemaphoreType.DMA,
            pltpu.SemaphoreType.DMA,
        ],
    )
    def gather_kernel(idx_hbm, table_hbm, out_hbm, ids_v, buf_a, buf_b,
                      tl_0, tl_1, out_0, out_1,
                      sem_ids, sem_row, sem_tail, sem_out):
        wid = lax.axis_index("s") * info.num_cores + lax.axis_index("c")
        bufs = (buf_a, buf_b)
        tails = (tl_0, tl_1)
        outs = (out_0, out_1)
        iota = lax.iota(jnp.int32, 16)

        def fire(rr, t, k):
            d = wid * rows_per_w + rr
            src = table_hbm.at[d, pl.ds(bases[t], sizes[t])]
            buf = bufs[k % 2]
            dst = buf if sizes[t] == _T else buf.at[pl.ds(0, sizes[t])]
            return pltpu.async_copy(src, dst, sem_row)

        def scan(rr, t, k):
            base, size, buf, ob = bases[t], sizes[t], bufs[k % 2], outs[rr]
            with_tail = t == 2
            tl = tails[rr]

            @plsc.parallel_loop(0, B // 16, unroll=4)
            def body(i):
                idxv = ids_v[pl.ds(i * 16, 16)]
                pos = iota + i * 16
                mask = jnp.logical_and(idxv >= base, idxv < base + size)
                vals = plsc.load_gather(buf, [idxv - base], mask=mask)
                plsc.store_scatter(ob, [pos], vals, mask=mask)
                if with_tail:
                    tmask = idxv >= main
                    tvals = plsc.load_gather(tl, [idxv - main], mask=tmask)
                    plsc.store_scatter(ob, [pos], tvals, mask=tmask)

        ids_cp = pltpu.async_copy(idx_hbm, ids_v, sem_ids)
        row_cps = [fire(0, 0, 0), fire(0, 1, 1)]
        tail_cps = [
            pltpu.async_copy(table_hbm.at[wid * rows_per_w + rr,
                                          pl.ds(main, tail)],
                             tails[rr], sem_tail)
            for rr in range(rows_per_w)
        ]
        ids_cp.wait()
        out_cps = []
        for rr in range(rows_per_w):
            d = wid * rows_per_w + rr
            for t in range(3):
                k = rr * 3 + t
                row_cps.pop(0).wait()
                if t == 2:
                    tail_cps[rr].wait()
                scan(rr, t, k)
                nxt = k + 2
                if nxt < rows_per_w * 3:
                    row_cps.append(fire(nxt // 3, nxt % 3, nxt))
            out_cps.append(
                pltpu.async_copy(outs[rr], out_hbm.at[d], sem_out))
        for cp in out_cps:
            cp.wait()

    return gather_kernel


def kernel(class_ids, table):
    (B,) = class_ids.shape
    V, D = table.shape
    gather_kernel = _build(B, V, D)
    out_t = gather_kernel(class_ids.astype(jnp.int32), table.T)
    return out_t.T


# speedup vs baseline: 1.1034x; 1.1034x over previous
import functools

import jax
import jax.numpy as jnp
from jax import lax
from jax.experimental import pallas as pl
from jax.experimental.pallas import tpu as pltpu
from jax.experimental.pallas import tpu_sc as plsc

_CHUNK = 4096


@functools.lru_cache(maxsize=None)
def _build(B, V, D):
    info = plsc.get_sparse_core_info()
    nw = info.num_cores * info.num_subcores
    rows_per_w = D // nw
    n_chunks = B // _CHUNK
    mesh = plsc.VectorSubcoreMesh(core_axis_name="c", subcore_axis_name="s")

    @functools.partial(
        pl.kernel,
        mesh=mesh,
        out_type=jax.ShapeDtypeStruct((D, B), jnp.float32),
        compiler_params=pltpu.CompilerParams(needs_layout_passes=False),
        scratch_types=[
            pltpu.VMEM((B,), jnp.int32),
            pltpu.VMEM((V,), jnp.float32),
            pltpu.VMEM((_CHUNK,), jnp.float32),
            pltpu.VMEM((_CHUNK,), jnp.float32),
            pltpu.SemaphoreType.DMA,
            pltpu.SemaphoreType.DMA,
            pltpu.SemaphoreType.DMA,
        ],
    )
    def gather_kernel(idx_hbm, table_hbm, out_hbm, ids_v, row_v, out_a, out_b,
                      sem_ids, sem_row, sem_out):
        out_bufs = (out_a, out_b)
        wid = lax.axis_index("s") * info.num_cores + lax.axis_index("c")
        ids_cp = pltpu.async_copy(idx_hbm, ids_v, sem_ids)
        row_cp = pltpu.async_copy(table_hbm.at[wid * rows_per_w], row_v, sem_row)
        ids_cp.wait()
        for rr in range(rows_per_w):
            d = wid * rows_per_w + rr
            row_cp.wait()
            out_cps = [None, None]
            for c in range(n_chunks):
                ob = out_bufs[c % 2]
                if out_cps[c % 2] is not None:
                    out_cps[c % 2].wait()

                @plsc.parallel_loop(0, _CHUNK // 16, unroll=8)
                def body(i):
                    idxv = ids_v[pl.ds(c * _CHUNK + i * 16, 16)]
                    ob[pl.ds(i * 16, 16)] = plsc.load_gather(row_v, [idxv])

                out_cps[c % 2] = pltpu.async_copy(
                    ob, out_hbm.at[d, pl.ds(c * _CHUNK, _CHUNK)], sem_out)
            if rr + 1 < rows_per_w:
                row_cp = pltpu.async_copy(table_hbm.at[d + 1], row_v, sem_row)
            for cp in out_cps:
                cp.wait()

    return gather_kernel


def kernel(class_ids, table):
    (B,) = class_ids.shape
    V, D = table.shape
    gather_kernel = _build(B, V, D)
    out_t = gather_kernel(class_ids.astype(jnp.int32), table.T)
    return out_t.T
